# Initial kernel scaffold; baseline (speedup 1.0000x reference)
import functools

import jax
import jax.numpy as jnp
from jax import lax
from jax.experimental import pallas as pl
from jax.experimental.pallas import tpu as pltpu
from jax.experimental.pallas import tpu_sc as plsc

N_NODES = 10000
N_EDGES = 10000
N_INC = 320000
F = 128

NC = 2
NS = 16
CHUNK = 80
ROWS_TOTAL = N_INC // CHUNK
ROWS_PER_TILE_HALF = ROWS_TOTAL // (NC * NS)
ROWS_PER_TILE_FULL = ROWS_TOTAL // NS
STRIPE = N_NODES // NS

_mesh = plsc.VectorSubcoreMesh(core_axis_name="c", subcore_axis_name="s")


@functools.partial(
    pl.kernel,
    out_type=jax.ShapeDtypeStruct((2, N_NODES), jnp.float32),
    mesh=_mesh,
    scratch_types=[
        pltpu.VMEM((ROWS_PER_TILE_FULL, CHUNK), jnp.int32),
        pltpu.VMEM((ROWS_PER_TILE_FULL, CHUNK), jnp.int32),
        pltpu.VMEM((CHUNK,), jnp.float32),
        pltpu.VMEM((CHUNK,), jnp.float32),
        pltpu.VMEM_SHARED((N_NODES,), jnp.float32),
        pltpu.SemaphoreType.DMA,
    ],
)
def _degrees_kernel(nidx_hbm, eidx_hbm, w_hbm, zeros1_hbm, out_hbm,
                    nidx_v, eidx_v, wval_v, ones_v, acc_sh, sem):
    cid = lax.axis_index("c")
    sid = lax.axis_index("s")

    @pl.when(sid == 0)
    def _zero():
        pltpu.sync_copy(zeros1_hbm, acc_sh)

    for i in range(CHUNK // 16):
        ones_v[pl.ds(i * 16, 16)] = jnp.full((16,), 1.0, jnp.float32)

    base = sid * ROWS_PER_TILE_FULL
    pltpu.sync_copy(nidx_hbm.at[pl.ds(base, ROWS_PER_TILE_FULL)], nidx_v)
    pltpu.sync_copy(eidx_hbm.at[pl.ds(base, ROWS_PER_TILE_FULL)], eidx_v)

    plsc.subcore_barrier()

    @pl.when(cid == 0)
    def _d_loop():
        def body(j, carry):
            pltpu.async_copy(w_hbm.at[eidx_v.at[j]], wval_v, sem).wait()
            pltpu.sync_copy(wval_v, acc_sh.at[nidx_v.at[j]], add=True)
            return carry
        lax.fori_loop(0, ROWS_PER_TILE_FULL, body, 0)

    @pl.when(cid == 1)
    def _b_loop():
        def body(j, carry):
            pltpu.sync_copy(ones_v, acc_sh.at[eidx_v.at[j]], add=True)
            return carry
        lax.fori_loop(0, ROWS_PER_TILE_FULL, body, 0)

    plsc.subcore_barrier()

    @pl.when(sid == 0)
    def _writeout():
        pltpu.sync_copy(acc_sh, out_hbm.at[cid])


@functools.partial(
    pl.kernel,
    out_type=jax.ShapeDtypeStruct((NC, N_NODES, F), jnp.float32),
    mesh=_mesh,
    scratch_types=[
        pltpu.VMEM((ROWS_PER_TILE_HALF, CHUNK), jnp.int32),
        pltpu.VMEM((ROWS_PER_TILE_HALF, CHUNK), jnp.int32),
        pltpu.VMEM((CHUNK, F), jnp.float32),
        pltpu.VMEM_SHARED((N_NODES, F), jnp.float32),
        pltpu.SemaphoreType.DMA,
    ],
)
def _row_phase_kernel(srcidx_hbm, dstidx_hbm, table_hbm, zeros2_hbm, out_hbm,
                      sidx_v, didx_v, rows_v, acc_sh, sem):
    cid = lax.axis_index("c")
    sid = lax.axis_index("s")

    pltpu.sync_copy(zeros2_hbm.at[pl.ds(sid * STRIPE, STRIPE)],
                    acc_sh.at[pl.ds(sid * STRIPE, STRIPE)])

    base = cid * (ROWS_TOTAL // NC) + sid * ROWS_PER_TILE_HALF
    pltpu.sync_copy(srcidx_hbm.at[pl.ds(base, ROWS_PER_TILE_HALF)], sidx_v)
    pltpu.sync_copy(dstidx_hbm.at[pl.ds(base, ROWS_PER_TILE_HALF)], didx_v)

    plsc.subcore_barrier()

    def body(j, carry):
        pltpu.async_copy(table_hbm.at[sidx_v.at[j]], rows_v, sem).wait()
        pltpu.sync_copy(rows_v, acc_sh.at[didx_v.at[j]], add=True)
        return carry
    lax.fori_loop(0, ROWS_PER_TILE_HALF, body, 0)

    plsc.subcore_barrier()

    pltpu.sync_copy(acc_sh.at[pl.ds(sid * STRIPE, STRIPE)],
                    out_hbm.at[cid, pl.ds(sid * STRIPE, STRIPE)])


def _matmul_body(x_ref, w_ref, o_ref):
    o_ref[...] = jnp.dot(x_ref[...], w_ref[...],
                         preferred_element_type=jnp.float32)


def _tc_matmul(x, w):
    return pl.pallas_call(
        _matmul_body,
        out_shape=jax.ShapeDtypeStruct((N_NODES, F), jnp.float32),
    )(x, w)


def _combine_body(p_ref, deg_ref, bias_ref, o_ref):
    d = deg_ref[...]
    inv = jnp.where(d > 0, 1.0 / jnp.where(d > 0, d, 1.0), 0.0)
    o_ref[...] = (p_ref[0] + p_ref[1]) * inv + bias_ref[...]


def _tc_combine(partials, deg, bias_row):
    return pl.pallas_call(
        _combine_body,
        out_shape=jax.ShapeDtypeStruct((N_NODES, F), jnp.float32),
    )(partials, deg, bias_row)


def kernel(x, hyperedge_index, hyperedge_weight, W_lin, bias):
    node_idx2d = hyperedge_index[0].astype(jnp.int32).reshape(ROWS_TOTAL, CHUNK)
    edge_idx2d = hyperedge_index[1].astype(jnp.int32).reshape(ROWS_TOTAL, CHUNK)
    zeros1 = jnp.zeros((N_NODES,), jnp.float32)
    zeros2 = jnp.zeros((N_NODES, F), jnp.float32)
    zero_bias = jnp.zeros((1, F), jnp.float32)

    xl = _tc_matmul(x, W_lin)

    degs = _degrees_kernel(node_idx2d, edge_idx2d,
                           hyperedge_weight.astype(jnp.float32), zeros1)
    d_col = degs[0][:, None]
    b_col = degs[1][:, None]

    p = _row_phase_kernel(node_idx2d, edge_idx2d, xl, zeros2)
    edge_out = _tc_combine(p, b_col, zero_bias)

    q = _row_phase_kernel(edge_idx2d, node_idx2d, edge_out, zeros2)
    return _tc_combine(q, d_col, bias[None, :].astype(jnp.float32))


# speedup vs baseline: 15.1124x; 15.1124x over previous
import functools

import jax
import jax.numpy as jnp
from jax import lax
from jax.experimental import pallas as pl
from jax.experimental.pallas import tpu as pltpu
from jax.experimental.pallas import tpu_sc as plsc

N_NODES = 10000
N_EDGES = 10000
N_INC = 320000
F = 128

NC = 2
NS = 16
CHUNK = 80
ROWS_TOTAL = N_INC // CHUNK
ROWS_PER_TILE_HALF = ROWS_TOTAL // (NC * NS)
ROWS_PER_TILE_FULL = ROWS_TOTAL // NS

STRIPE = 632
STRIPE_LAST_OFF = (NS - 1) * STRIPE
STRIPE_LAST = N_NODES - STRIPE_LAST_OFF

_mesh = plsc.VectorSubcoreMesh(core_axis_name="c", subcore_axis_name="s")


@functools.partial(
    pl.kernel,
    out_type=(jax.ShapeDtypeStruct((N_NODES,), jnp.float32),
              jax.ShapeDtypeStruct((N_NODES,), jnp.float32)),
    mesh=_mesh,
    scratch_types=[
        pltpu.VMEM((ROWS_PER_TILE_FULL, CHUNK), jnp.int32),
        pltpu.VMEM((ROWS_PER_TILE_FULL, CHUNK), jnp.int32),
        pltpu.VMEM((CHUNK,), jnp.float32),
        pltpu.VMEM((CHUNK,), jnp.float32),
        pltpu.VMEM_SHARED((N_NODES,), jnp.float32),
        pltpu.SemaphoreType.DMA,
    ],
)
def _degrees_kernel(nidx_hbm, eidx_hbm, w_hbm, zeros1_hbm, d_out_hbm, b_out_hbm,
                    nidx_v, eidx_v, wval_v, ones_v, acc_sh, sem):
    cid = lax.axis_index("c")
    sid = lax.axis_index("s")

    @pl.when(sid == 0)
    def _zero():
        pltpu.sync_copy(zeros1_hbm, acc_sh)

    for i in range(CHUNK // 16):
        ones_v[pl.ds(i * 16, 16)] = jnp.full((16,), 1.0, jnp.float32)

    pltpu.sync_copy(nidx_hbm.at[sid], nidx_v)
    pltpu.sync_copy(eidx_hbm.at[sid], eidx_v)

    plsc.subcore_barrier()

    @pl.when(cid == 0)
    def _d_loop():
        def body(j, carry):
            pltpu.async_copy(w_hbm.at[eidx_v.at[j]], wval_v, sem).wait()
            pltpu.sync_copy(wval_v, acc_sh.at[nidx_v.at[j]], add=True)
            return carry
        lax.fori_loop(0, ROWS_PER_TILE_FULL, body, 0)

    @pl.when(cid == 1)
    def _b_loop():
        def body(j, carry):
            pltpu.sync_copy(ones_v, acc_sh.at[eidx_v.at[j]], add=True)
            return carry
        lax.fori_loop(0, ROWS_PER_TILE_FULL, body, 0)

    plsc.subcore_barrier()

    @pl.when(jnp.logical_and(sid == 0, cid == 0))
    def _write_d():
        pltpu.sync_copy(acc_sh, d_out_hbm)

    @pl.when(jnp.logical_and(sid == 0, cid == 1))
    def _write_b():
        pltpu.sync_copy(acc_sh, b_out_hbm)


@functools.partial(
    pl.kernel,
    out_type=jax.ShapeDtypeStruct((NC, N_NODES, F), jnp.float32),
    mesh=_mesh,
    scratch_types=[
        pltpu.VMEM((ROWS_PER_TILE_HALF, CHUNK), jnp.int32),
        pltpu.VMEM((ROWS_PER_TILE_HALF, CHUNK), jnp.int32),
        pltpu.VMEM((CHUNK, F), jnp.float32),
        pltpu.VMEM_SHARED((N_NODES, F), jnp.float32),
        pltpu.SemaphoreType.DMA,
    ],
)
def _row_phase_kernel(srcidx_hbm, dstidx_hbm, table_hbm, zeros2_hbm, out_hbm,
                      sidx_v, didx_v, rows_v, acc_sh, sem):
    cid = lax.axis_index("c")
    sid = lax.axis_index("s")
    wid = cid * NS + sid

    @pl.when(sid < NS - 1)
    def _zero_a():
        off = pl.multiple_of(sid * STRIPE, 8)
        pltpu.sync_copy(zeros2_hbm.at[pl.ds(off, STRIPE)],
                        acc_sh.at[pl.ds(off, STRIPE)])

    @pl.when(sid == NS - 1)
    def _zero_b():
        pltpu.sync_copy(zeros2_hbm.at[pl.ds(STRIPE_LAST_OFF, STRIPE_LAST)],
                        acc_sh.at[pl.ds(STRIPE_LAST_OFF, STRIPE_LAST)])

    pltpu.sync_copy(srcidx_hbm.at[wid], sidx_v)
    pltpu.sync_copy(dstidx_hbm.at[wid], didx_v)

    plsc.subcore_barrier()

    def body(j, carry):
        pltpu.async_copy(table_hbm.at[sidx_v.at[j]], rows_v, sem).wait()
        pltpu.sync_copy(rows_v, acc_sh.at[didx_v.at[j]], add=True)
        return carry
    lax.fori_loop(0, ROWS_PER_TILE_HALF, body, 0)

    plsc.subcore_barrier()

    @pl.when(sid < NS - 1)
    def _write_a():
        off = pl.multiple_of(sid * STRIPE, 8)
        pltpu.sync_copy(acc_sh.at[pl.ds(off, STRIPE)],
                        out_hbm.at[cid, pl.ds(off, STRIPE)])

    @pl.when(sid == NS - 1)
    def _write_b():
        pltpu.sync_copy(acc_sh.at[pl.ds(STRIPE_LAST_OFF, STRIPE_LAST)],
                        out_hbm.at[cid, pl.ds(STRIPE_LAST_OFF, STRIPE_LAST)])


def _matmul_body(x_ref, w_ref, o_ref):
    o_ref[...] = jnp.dot(x_ref[...], w_ref[...],
                         preferred_element_type=jnp.float32)


def _tc_matmul(x, w):
    return pl.pallas_call(
        _matmul_body,
        out_shape=jax.ShapeDtypeStruct((N_NODES, F), jnp.float32),
    )(x, w)


def _combine_body(p_ref, deg_ref, bias_ref, o_ref):
    d = deg_ref[...]
    inv = jnp.where(d > 0, 1.0 / jnp.where(d > 0, d, 1.0), 0.0)
    o_ref[...] = (p_ref[0] + p_ref[1]) * inv + bias_ref[...]


def _tc_combine(partials, deg, bias_row):
    return pl.pallas_call(
        _combine_body,
        out_shape=jax.ShapeDtypeStruct((N_NODES, F), jnp.float32),
    )(partials, deg, bias_row)


def kernel(x, hyperedge_index, hyperedge_weight, W_lin, bias):
    node_idx = hyperedge_index[0].astype(jnp.int32)
    edge_idx = hyperedge_index[1].astype(jnp.int32)
    nidx_f = node_idx.reshape(NS, ROWS_PER_TILE_FULL, CHUNK)
    eidx_f = edge_idx.reshape(NS, ROWS_PER_TILE_FULL, CHUNK)
    nidx_h = node_idx.reshape(NC * NS, ROWS_PER_TILE_HALF, CHUNK)
    eidx_h = edge_idx.reshape(NC * NS, ROWS_PER_TILE_HALF, CHUNK)
    zeros1 = jnp.zeros((N_NODES,), jnp.float32)
    zeros2 = jnp.zeros((N_NODES, F), jnp.float32)
    zero_bias = jnp.zeros((1, F), jnp.float32)

    xl = _tc_matmul(x, W_lin)

    d_deg, b_deg = _degrees_kernel(nidx_f, eidx_f,
                                   hyperedge_weight.astype(jnp.float32), zeros1)
    d_col = d_deg[:, None]
    b_col = b_deg[:, None]

    p = _row_phase_kernel(nidx_h, eidx_h, xl, zeros2)
    edge_out = _tc_combine(p, b_col, zero_bias)

    q = _row_phase_kernel(eidx_h, nidx_h, edge_out, zeros2)
    return _tc_combine(q, d_col, bias[None, :].astype(jnp.float32))


# speedup vs baseline: 30.8077x; 2.0386x over previous
import functools

import jax
import jax.numpy as jnp
from jax import lax
from jax.experimental import pallas as pl
from jax.experimental.pallas import tpu as pltpu
from jax.experimental.pallas import tpu_sc as plsc

N_NODES = 10000
N_EDGES = 10000
N_INC = 320000
F = 128

NC = 2
NS = 16
CHUNK = 125
ROWS_TOTAL = N_INC // CHUNK
ROWS_PER_TILE_HALF = ROWS_TOTAL // (NC * NS)
BLK = 16
NBLK = ROWS_PER_TILE_HALF // BLK

STRIPE = 632
STRIPE_LAST_OFF = (NS - 1) * STRIPE
STRIPE_LAST = N_NODES - STRIPE_LAST_OFF

_mesh = plsc.VectorSubcoreMesh(core_axis_name="c", subcore_axis_name="s")


@functools.partial(
    pl.kernel,
    out_type=(jax.ShapeDtypeStruct((N_NODES,), jnp.float32),
              jax.ShapeDtypeStruct((N_NODES,), jnp.float32),
              jax.ShapeDtypeStruct((N_NODES,), jnp.float32),
              jax.ShapeDtypeStruct((N_NODES,), jnp.float32)),
    mesh=_mesh,
    scratch_types=[
        pltpu.VMEM((ROWS_PER_TILE_HALF, CHUNK), jnp.int32),
        pltpu.VMEM((ROWS_PER_TILE_HALF, CHUNK), jnp.int32),
        pltpu.VMEM((2, CHUNK), jnp.float32),
        pltpu.VMEM((CHUNK,), jnp.float32),
        pltpu.VMEM_SHARED((N_NODES,), jnp.float32),
        pltpu.VMEM_SHARED((N_NODES,), jnp.float32),
        pltpu.SemaphoreType.DMA,
        pltpu.SemaphoreType.DMA,
    ],
)
def _degrees_kernel(nidx_hbm, eidx_hbm, w_hbm, zeros1_hbm,
                    d0_out, d1_out, b0_out, b1_out,
                    nidx_v, eidx_v, wval_v, ones_v, dacc_sh, bacc_sh,
                    semg, semb):
    cid = lax.axis_index("c")
    sid = lax.axis_index("s")
    wid = cid * NS + sid

    @pl.when(sid == 0)
    def _zero_d():
        pltpu.sync_copy(zeros1_hbm, dacc_sh)

    @pl.when(sid == 1)
    def _zero_b():
        pltpu.sync_copy(zeros1_hbm, bacc_sh)

    for i in range((CHUNK + 15) // 16):
        ones_v[pl.ds(min(16 * i, CHUNK - 16), 16)] = jnp.full(
            (16,), 1.0, jnp.float32)

    pltpu.sync_copy(nidx_hbm.at[wid], nidx_v)
    pltpu.sync_copy(eidx_hbm.at[wid], eidx_v)

    plsc.subcore_barrier()

    pltpu.async_copy(w_hbm.at[eidx_v.at[0]], wval_v.at[0], semg)

    def body(j, carry):
        @pl.when(j < ROWS_PER_TILE_HALF - 1)
        def _fire_next():
            pltpu.async_copy(w_hbm.at[eidx_v.at[j + 1]],
                             wval_v.at[(j + 1) % 2], semg)
        pltpu.make_async_copy(w_hbm.at[eidx_v.at[0]],
                              wval_v.at[j % 2], semg).wait()
        pltpu.sync_copy(wval_v.at[j % 2], dacc_sh.at[nidx_v.at[j]], add=True)
        pltpu.async_copy(ones_v, bacc_sh.at[eidx_v.at[j]], semb, add=True)
        return carry
    lax.fori_loop(0, ROWS_PER_TILE_HALF, body, 0)

    def drain(j, carry):
        pltpu.make_async_copy(ones_v, bacc_sh.at[eidx_v.at[0]], semb).wait()
        return carry
    lax.fori_loop(0, ROWS_PER_TILE_HALF, drain, 0)

    plsc.subcore_barrier()

    @pl.when(jnp.logical_and(sid == 0, cid == 0))
    def _write_d0():
        pltpu.sync_copy(dacc_sh, d0_out)

    @pl.when(jnp.logical_and(sid == 0, cid == 1))
    def _write_d1():
        pltpu.sync_copy(dacc_sh, d1_out)

    @pl.when(jnp.logical_and(sid == 1, cid == 0))
    def _write_b0():
        pltpu.sync_copy(bacc_sh, b0_out)

    @pl.when(jnp.logical_and(sid == 1, cid == 1))
    def _write_b1():
        pltpu.sync_copy(bacc_sh, b1_out)


@functools.partial(
    pl.kernel,
    out_type=jax.ShapeDtypeStruct((NC, N_NODES, F), jnp.float32),
    mesh=_mesh,
    scratch_types=[
        pltpu.VMEM((2, BLK, CHUNK), jnp.int32),
        pltpu.VMEM((2, BLK, CHUNK), jnp.int32),
        pltpu.VMEM((2, CHUNK, F), jnp.float32),
        pltpu.VMEM_SHARED((N_NODES, F), jnp.float32),
        pltpu.SemaphoreType.DMA,
        pltpu.SemaphoreType.DMA,
    ],
)
def _row_phase_kernel(srcidx_hbm, dstidx_hbm, table_hbm, zeros2_hbm, out_hbm,
                      sidx_v, didx_v, rows_v, acc_sh, semg, semi):
    cid = lax.axis_index("c")
    sid = lax.axis_index("s")
    wid = cid * NS + sid

    @pl.when(sid < NS - 1)
    def _zero_a():
        off = pl.multiple_of(sid * STRIPE, 8)
        pltpu.sync_copy(zeros2_hbm.at[pl.ds(off, STRIPE)],
                        acc_sh.at[pl.ds(off, STRIPE)])

    @pl.when(sid == NS - 1)
    def _zero_b():
        pltpu.sync_copy(zeros2_hbm.at[pl.ds(STRIPE_LAST_OFF, STRIPE_LAST)],
                        acc_sh.at[pl.ds(STRIPE_LAST_OFF, STRIPE_LAST)])

    pltpu.async_copy(srcidx_hbm.at[wid, pl.ds(0, BLK)], sidx_v.at[0], semi)
    pltpu.async_copy(dstidx_hbm.at[wid, pl.ds(0, BLK)], didx_v.at[0], semi)

    plsc.subcore_barrier()

    def outer(b, carry):
        pb = b % 2
        pltpu.make_async_copy(srcidx_hbm.at[wid, pl.ds(0, BLK)],
                              sidx_v.at[pb], semi).wait()
        pltpu.make_async_copy(dstidx_hbm.at[wid, pl.ds(0, BLK)],
                              didx_v.at[pb], semi).wait()

        @pl.when(b < NBLK - 1)
        def _fire_next_block():
            off = pl.multiple_of((b + 1) * BLK, 8)
            pltpu.async_copy(srcidx_hbm.at[wid, pl.ds(off, BLK)],
                             sidx_v.at[(b + 1) % 2], semi)
            pltpu.async_copy(dstidx_hbm.at[wid, pl.ds(off, BLK)],
                             didx_v.at[(b + 1) % 2], semi)

        pltpu.async_copy(table_hbm.at[sidx_v.at[pb, 0]], rows_v.at[0], semg)

        def inner(j, c2):
            @pl.when(j < BLK - 1)
            def _fire_next():
                pltpu.async_copy(table_hbm.at[sidx_v.at[pb, j + 1]],
                                 rows_v.at[(j + 1) % 2], semg)
            pltpu.make_async_copy(table_hbm.at[sidx_v.at[pb, 0]],
                                  rows_v.at[j % 2], semg).wait()
            pltpu.sync_copy(rows_v.at[j % 2],
                            acc_sh.at[didx_v.at[pb, j]], add=True)
            return c2
        lax.fori_loop(0, BLK, inner, 0)
        return carry
    lax.fori_loop(0, NBLK, outer, 0)

    plsc.subcore_barrier()

    @pl.when(sid < NS - 1)
    def _write_a():
        off = pl.multiple_of(sid * STRIPE, 8)
        pltpu.sync_copy(acc_sh.at[pl.ds(off, STRIPE)],
                        out_hbm.at[cid, pl.ds(off, STRIPE)])

    @pl.when(sid == NS - 1)
    def _write_b():
        pltpu.sync_copy(acc_sh.at[pl.ds(STRIPE_LAST_OFF, STRIPE_LAST)],
                        out_hbm.at[cid, pl.ds(STRIPE_LAST_OFF, STRIPE_LAST)])


def _matmul_body(x_ref, w_ref, o_ref):
    o_ref[...] = jnp.dot(x_ref[...], w_ref[...],
                         preferred_element_type=jnp.float32)


def _tc_matmul(x, w):
    return pl.pallas_call(
        _matmul_body,
        out_shape=jax.ShapeDtypeStruct((N_NODES, F), jnp.float32),
    )(x, w)


def _combine_body(p_ref, dega_ref, degb_ref, bias_ref, o_ref):
    d = dega_ref[...] + degb_ref[...]
    inv = jnp.where(d > 0, 1.0 / jnp.where(d > 0, d, 1.0), 0.0)
    o_ref[...] = (p_ref[0] + p_ref[1]) * inv + bias_ref[...]


def _tc_combine(partials, dega, degb, bias_row):
    return pl.pallas_call(
        _combine_body,
        out_shape=jax.ShapeDtypeStruct((N_NODES, F), jnp.float32),
    )(partials, dega, degb, bias_row)


def kernel(x, hyperedge_index, hyperedge_weight, W_lin, bias):
    node_idx = hyperedge_index[0].astype(jnp.int32)
    edge_idx = hyperedge_index[1].astype(jnp.int32)
    nidx_h = node_idx.reshape(NC * NS, ROWS_PER_TILE_HALF, CHUNK)
    eidx_h = edge_idx.reshape(NC * NS, ROWS_PER_TILE_HALF, CHUNK)
    zeros1 = jnp.zeros((N_NODES,), jnp.float32)
    zeros2 = jnp.zeros((N_NODES, F), jnp.float32)
    zero_bias = jnp.zeros((1, F), jnp.float32)

    xl = _tc_matmul(x, W_lin)

    d0, d1, b0, b1 = _degrees_kernel(nidx_h, eidx_h,
                                     hyperedge_weight.astype(jnp.float32),
                                     zeros1)

    p = _row_phase_kernel(nidx_h, eidx_h, xl, zeros2)
    edge_out = _tc_combine(p, b0[:, None], b1[:, None], zero_bias)

    q = _row_phase_kernel(eidx_h, nidx_h, edge_out, zeros2)
    return _tc_combine(q, d0[:, None], d1[:, None],
                       bias[None, :].astype(jnp.float32))


# speedup vs baseline: 33.5390x; 1.0887x over previous
import jax
import jax.numpy as jnp
from jax import lax
from jax.experimental import pallas as pl
from jax.experimental.pallas import tpu as pltpu
from jax.experimental.pallas import tpu_sc as plsc

N_NODES = 10000
N_EDGES = 10000
N_INC = 320000
F = 128

NC = 2
NS = 16
CHUNK = 125
ROWS_TOTAL = N_INC // CHUNK
ROWS_PER_TILE = ROWS_TOTAL // (NC * NS)
BLK = 16
NBLK = ROWS_PER_TILE // BLK

STRIPE = 632
STRIPE_LAST_OFF = (NS - 1) * STRIPE
STRIPE_LAST = N_NODES - STRIPE_LAST_OFF

_mesh = plsc.VectorSubcoreMesh(core_axis_name="c", subcore_axis_name="s")


def _build_row_phase(with_degrees):
    outs = (jax.ShapeDtypeStruct((NC, N_NODES, F), jnp.float32),)
    scratch = [
        pltpu.VMEM((2, BLK, CHUNK), jnp.int32),
        pltpu.VMEM((2, BLK, CHUNK), jnp.int32),
        pltpu.VMEM((2, CHUNK, F), jnp.float32),
        pltpu.VMEM_SHARED((N_NODES, F), jnp.float32),
        pltpu.SemaphoreType.DMA,
        pltpu.SemaphoreType.DMA,
    ]
    if with_degrees:
        outs = outs + (jax.ShapeDtypeStruct((N_NODES,), jnp.float32),) * 4
        scratch += [
            pltpu.VMEM((BLK, CHUNK), jnp.float32),
            pltpu.VMEM((CHUNK,), jnp.float32),
            pltpu.VMEM_SHARED((N_NODES,), jnp.float32),
            pltpu.VMEM_SHARED((N_NODES,), jnp.float32),
            pltpu.SemaphoreType.DMA,
            pltpu.SemaphoreType.DMA,
            pltpu.SemaphoreType.DMA,
        ]

    def body(*refs):
        if with_degrees:
            (srcidx_hbm, dstidx_hbm, table_hbm, zeros2_hbm, w_hbm, zeros1_hbm,
             out_hbm, d0_out, d1_out, b0_out, b1_out,
             sidx_v, didx_v, rows_v, acc_sh, semg, semi,
             wval_v, ones_v, dacc_sh, bacc_sh, semw, semd, semb) = refs
        else:
            (srcidx_hbm, dstidx_hbm, table_hbm, zeros2_hbm, out_hbm,
             sidx_v, didx_v, rows_v, acc_sh, semg, semi) = refs

        cid = lax.axis_index("c")
        sid = lax.axis_index("s")
        wid = cid * NS + sid

        @pl.when(sid < NS - 1)
        def _zero_a():
            off = pl.multiple_of(sid * STRIPE, 8)
            pltpu.sync_copy(zeros2_hbm.at[pl.ds(off, STRIPE)],
                            acc_sh.at[pl.ds(off, STRIPE)])

        @pl.when(sid == NS - 1)
        def _zero_b():
            pltpu.sync_copy(zeros2_hbm.at[pl.ds(STRIPE_LAST_OFF, STRIPE_LAST)],
                            acc_sh.at[pl.ds(STRIPE_LAST_OFF, STRIPE_LAST)])

        if with_degrees:
            @pl.when(sid == 0)
            def _zero_d():
                pltpu.sync_copy(zeros1_hbm, dacc_sh)

            @pl.when(sid == 1)
            def _zero_bdeg():
                pltpu.sync_copy(zeros1_hbm, bacc_sh)

            for i in range((CHUNK + 15) // 16):
                ones_v[pl.ds(min(16 * i, CHUNK - 16), 16)] = jnp.full(
                    (16,), 1.0, jnp.float32)

        pltpu.async_copy(srcidx_hbm.at[wid, pl.ds(0, BLK)], sidx_v.at[0], semi)
        pltpu.async_copy(dstidx_hbm.at[wid, pl.ds(0, BLK)], didx_v.at[0], semi)

        plsc.subcore_barrier()

        def outer(b, carry):
            pb = b % 2
            pltpu.make_async_copy(srcidx_hbm.at[wid, pl.ds(0, BLK)],
                                  sidx_v.at[pb], semi).wait()
            pltpu.make_async_copy(dstidx_hbm.at[wid, pl.ds(0, BLK)],
                                  didx_v.at[pb], semi).wait()

            @pl.when(b < NBLK - 1)
            def _fire_next_block():
                off = pl.multiple_of((b + 1) * BLK, 8)
                pltpu.async_copy(srcidx_hbm.at[wid, pl.ds(off, BLK)],
                                 sidx_v.at[(b + 1) % 2], semi)
                pltpu.async_copy(dstidx_hbm.at[wid, pl.ds(off, BLK)],
                                 didx_v.at[(b + 1) % 2], semi)

            pltpu.async_copy(table_hbm.at[sidx_v.at[pb, 0]], rows_v.at[0],
                             semg)
            if with_degrees:
                pltpu.async_copy(w_hbm.at[didx_v.at[pb, 0]], wval_v.at[0],
                                 semw)

            def inner(j, c2):
                @pl.when(j < BLK - 1)
                def _fire_next():
                    pltpu.async_copy(table_hbm.at[sidx_v.at[pb, j + 1]],
                                     rows_v.at[(j + 1) % 2], semg)
                    if with_degrees:
                        pltpu.async_copy(w_hbm.at[didx_v.at[pb, j + 1]],
                                         wval_v.at[j + 1], semw)
                pltpu.make_async_copy(table_hbm.at[sidx_v.at[pb, 0]],
                                      rows_v.at[j % 2], semg).wait()
                pltpu.sync_copy(rows_v.at[j % 2],
                                acc_sh.at[didx_v.at[pb, j]], add=True)
                if with_degrees:
                    pltpu.make_async_copy(w_hbm.at[didx_v.at[pb, 0]],
                                          wval_v.at[0], semw).wait()
                    pltpu.async_copy(wval_v.at[j],
                                     dacc_sh.at[sidx_v.at[pb, j]], semd,
                                     add=True)
                    pltpu.async_copy(ones_v, bacc_sh.at[didx_v.at[pb, j]],
                                     semb, add=True)
                return c2
            lax.fori_loop(0, BLK, inner, 0)

            if with_degrees:
                def draind(j, c3):
                    pltpu.make_async_copy(
                        wval_v.at[0], dacc_sh.at[sidx_v.at[0, 0]],
                        semd).wait()
                    return c3
                lax.fori_loop(0, BLK, draind, 0)
            return carry
        lax.fori_loop(0, NBLK, outer, 0)

        if with_degrees:
            def drainb(j, c4):
                pltpu.make_async_copy(ones_v, bacc_sh.at[didx_v.at[0, 0]],
                                      semb).wait()
                return c4
            lax.fori_loop(0, ROWS_PER_TILE, drainb, 0)

        plsc.subcore_barrier()

        @pl.when(sid < NS - 1)
        def _write_a():
            off = pl.multiple_of(sid * STRIPE, 8)
            pltpu.sync_copy(acc_sh.at[pl.ds(off, STRIPE)],
                            out_hbm.at[cid, pl.ds(off, STRIPE)])

        @pl.when(sid == NS - 1)
        def _write_b():
            pltpu.sync_copy(acc_sh.at[pl.ds(STRIPE_LAST_OFF, STRIPE_LAST)],
                            out_hbm.at[cid, pl.ds(STRIPE_LAST_OFF,
                                                  STRIPE_LAST)])

        if with_degrees:
            @pl.when(jnp.logical_and(sid == 0, cid == 0))
            def _write_d0():
                pltpu.sync_copy(dacc_sh, d0_out)

            @pl.when(jnp.logical_and(sid == 0, cid == 1))
            def _write_d1():
                pltpu.sync_copy(dacc_sh, d1_out)

            @pl.when(jnp.logical_and(sid == 1, cid == 0))
            def _write_b0():
                pltpu.sync_copy(bacc_sh, b0_out)

            @pl.when(jnp.logical_and(sid == 1, cid == 1))
            def _write_b1():
                pltpu.sync_copy(bacc_sh, b1_out)

    return pl.kernel(body, out_type=outs, mesh=_mesh, scratch_types=scratch)


_row_phase_deg = _build_row_phase(True)
_row_phase = _build_row_phase(False)


def _matmul_body(x_ref, w_ref, o_ref):
    o_ref[...] = jnp.dot(x_ref[...], w_ref[...],
                         preferred_element_type=jnp.float32)


def _tc_matmul(x, w):
    return pl.pallas_call(
        _matmul_body,
        out_shape=jax.ShapeDtypeStruct((N_NODES, F), jnp.float32),
    )(x, w)


def _combine_body(p_ref, dega_ref, degb_ref, bias_ref, o_ref):
    d = dega_ref[...] + degb_ref[...]
    inv = jnp.where(d > 0, 1.0 / jnp.where(d > 0, d, 1.0), 0.0)
    o_ref[...] = (p_ref[0] + p_ref[1]) * inv + bias_ref[...]


def _tc_combine(partials, dega, degb, bias_row):
    return pl.pallas_call(
        _combine_body,
        out_shape=jax.ShapeDtypeStruct((N_NODES, F), jnp.float32),
    )(partials, dega, degb, bias_row)


def kernel(x, hyperedge_index, hyperedge_weight, W_lin, bias):
    node_idx = hyperedge_index[0].astype(jnp.int32)
    edge_idx = hyperedge_index[1].astype(jnp.int32)
    nidx = node_idx.reshape(NC * NS, ROWS_PER_TILE, CHUNK)
    eidx = edge_idx.reshape(NC * NS, ROWS_PER_TILE, CHUNK)
    zeros1 = jnp.zeros((N_NODES,), jnp.float32)
    zeros2 = jnp.zeros((N_NODES, F), jnp.float32)
    zero_bias = jnp.zeros((1, F), jnp.float32)

    xl = _tc_matmul(x, W_lin)

    p, d0, d1, b0, b1 = _row_phase_deg(
        nidx, eidx, xl, zeros2, hyperedge_weight.astype(jnp.float32), zeros1)
    edge_out = _tc_combine(p, b0[:, None], b1[:, None], zero_bias)

    (q,) = _row_phase(eidx, nidx, edge_out, zeros2)
    return _tc_combine(q, d0[:, None], d1[:, None],
                       bias[None, :].astype(jnp.float32))
